# baseline (speedup 1.0000x reference)
import functools

import jax
import jax.numpy as jnp
from jax import lax
from jax.experimental import pallas as pl
from jax.experimental.pallas import tpu as pltpu
from jax.experimental.pallas import tpu_sc as plsc

D_MODEL = 64
ROWS = 128
TSTRIDE = 129
TSIZE = D_MODEL * TSTRIDE


@functools.lru_cache(maxsize=None)
def _make_gather(vocab: int, n_j: int, n_i: int):
    info = plsc.get_sparse_core_info()
    num_workers = info.num_cores * info.num_subcores
    assert n_i == num_workers * ROWS and n_j % 2 == 0 and n_j >= 6

    mesh = plsc.VectorSubcoreMesh(core_axis_name="c", subcore_axis_name="s")

    @functools.partial(
        pl.kernel,
        mesh=mesh,
        compiler_params=pltpu.CompilerParams(needs_layout_passes=False),
        out_type=jax.ShapeDtypeStruct((n_j, D_MODEL, n_i), jnp.float32),
        scratch_types=[
            pltpu.VMEM((n_j, ROWS), jnp.int32),
            pltpu.VMEM((ROWS, D_MODEL), jnp.float32),
            pltpu.VMEM((ROWS, D_MODEL), jnp.float32),
            pltpu.VMEM((TSIZE,), jnp.float32),
            pltpu.VMEM((TSIZE,), jnp.float32),
            pltpu.VMEM((D_MODEL, ROWS), jnp.float32),
            pltpu.VMEM((D_MODEL, ROWS), jnp.float32),
            pltpu.SemaphoreType.DMA,
            pltpu.SemaphoreType.DMA,
            pltpu.SemaphoreType.DMA,
            pltpu.SemaphoreType.DMA,
        ],
    )
    def gather_kernel(idx_hbm, table_hbm, out_hbm, idx_v,
                      gr0, gr1, t1d0, t1d1, t2d0, t2d1,
                      gs0, gs1, ws0, ws1):
        wid = lax.axis_index("s") * info.num_cores + lax.axis_index("c")
        base = wid * ROWS
        grow = (gr0, gr1)
        t1d = (t1d0, t1d1)
        t2d = (t2d0, t2d1)
        gsem = (gs0, gs1)
        wsem = (ws0, ws1)

        pltpu.sync_copy(idx_hbm.at[:, pl.ds(base, ROWS)], idx_v)

        iota = lax.iota(jnp.int32, 16)
        sbase = tuple(iota * TSTRIDE + 16 * k * TSTRIDE for k in range(4))
        cbase = tuple(iota + 16 * k for k in range(4))

        def fire_gather(j, b):
            def grp(g, carry):
                ivec = idx_v.at[j][pl.ds(g * 16, 16)]
                for l in range(16):
                    pltpu.async_copy(
                        table_hbm.at[pl.ds(ivec[l], 1)],
                        grow[b].at[pl.ds(g * 16 + l, 1)],
                        gsem[b],
                    )
                return carry
            lax.fori_loop(0, ROWS // 16, grp, 0)

        def merged(j, bt, fire, compact):
            bn = 1 - bt

            def grp(g, carry):
                if fire:
                    ivec = idx_v.at[j + 1][pl.ds(g * 16, 16)]
                for l in range(16):
                    i = g * 16 + l
                    if fire:
                        pltpu.async_copy(
                            table_hbm.at[pl.ds(ivec[l], 1)],
                            grow[bn].at[pl.ds(i, 1)],
                            gsem[bn],
                        )
                    row = grow[bt].at[i]
                    for k in range(4):
                        vec = row[pl.ds(16 * k, 16)]
                        plsc.store_scatter(t1d[bt], [sbase[k] + i], vec)
                    if compact:
                        c = i // 2
                        half = (i % 2) * 64
                        off = c * TSTRIDE + half
                        for k in range(4):
                            v = plsc.load_gather(t1d[bn], [cbase[k] + off])
                            t2d[bn].at[c][pl.ds(half + 16 * k, 16)] = v
                return carry

            lax.fori_loop(0, ROWS // 16, grp, 0)

        def wait_gather(b):
            pltpu.make_async_copy(
                table_hbm.at[pl.ds(0, ROWS)], grow[b], gsem[b]
            ).wait()

        def fire_wb(j, b):
            pltpu.async_copy(
                t2d[b], out_hbm.at[j, :, pl.ds(base, ROWS)], wsem[b]
            )

        def wait_wb(j, b):
            pltpu.make_async_copy(
                t2d[b], out_hbm.at[j, :, pl.ds(base, ROWS)], wsem[b]
            ).wait()

        def compact_only(j, b):
            def grp(g, carry):
                for l in range(16):
                    i = g * 16 + l
                    c = i // 2
                    half = (i % 2) * 64
                    off = c * TSTRIDE + half
                    for k in range(4):
                        v = plsc.load_gather(t1d[b], [cbase[k] + off])
                        t2d[b].at[c][pl.ds(half + 16 * k, 16)] = v
                return carry
            lax.fori_loop(0, ROWS // 16, grp, 0)

        fire_gather(0, 0)
        wait_gather(0)
        merged(0, 0, fire=True, compact=False)
        wait_gather(1)
        merged(1, 1, fire=True, compact=True)
        fire_wb(0, 0)
        wait_gather(0)
        merged(2, 0, fire=True, compact=True)
        fire_wb(1, 1)

        def body(k, carry):
            for m in range(2):
                t = 3 + 2 * k + m
                bt = (3 + m) % 2
                wait_gather(bt)
                wait_wb(t - 3, m % 2)
                merged(t, bt, fire=True, compact=True)
                fire_wb(t - 1, m % 2)
            return carry

        lax.fori_loop(0, (n_j - 4) // 2, body, 0)

        t = n_j - 1
        wait_gather(t % 2)
        wait_wb(t - 3, (t - 3) % 2)
        merged(t, t % 2, fire=False, compact=True)
        fire_wb(t - 1, (t - 1) % 2)
        wait_wb(t - 2, t % 2)
        compact_only(t, t % 2)
        fire_wb(t, t % 2)
        wait_wb(t - 1, (t - 1) % 2)
        wait_wb(t, t % 2)

    return gather_kernel


def kernel(x, table):
    n_i, n_j = x.shape
    xt = jnp.transpose(x, (1, 0)).astype(jnp.int32)
    out = _make_gather(table.shape[0], n_j, n_i)(xt, table)
    return jnp.transpose(out, (2, 0, 1))
